# baseline (speedup 1.0000x reference)
import numpy as np

import jax
import jax.numpy as jnp
from jax import lax
from jax.experimental import pallas as pl
from jax.experimental.pallas import tpu as pltpu
from jax.experimental.pallas import tpu_sc as plsc

N_ROWS = 16384
D = 64
C = 64
MARGIN_ = 2.0

NUM_CORES = 2
NUM_SUBCORES = 16
NW = NUM_CORES * NUM_SUBCORES
RPW = N_ROWS // NW
L = 16
GROUPS = RPW // L

_RSQRT_MAGIC = np.int32(0x5F3759DF)


def _rsqrt16(s):
    bits = plsc.bitcast(s, jnp.int32)
    y = plsc.bitcast(_RSQRT_MAGIC - (bits >> 1), jnp.float32)
    for _ in range(3):
        y = y * (1.5 - 0.5 * s * y * y)
    return y


def _sc_body(fT, labels, sums_out, cnt_out, sq_out, fvm, lvm, acc, cnt2, sqv, idxm):
    cid = lax.axis_index("c")
    sid = lax.axis_index("s")
    wid = sid * NUM_CORES + cid

    pltpu.sync_copy(fT.at[:, pl.ds(wid * RPW, RPW)], fvm)
    pltpu.sync_copy(labels.at[pl.ds(wid * RPW, RPW)], lvm)

    zeros = jnp.zeros((L,), jnp.float32)
    ones = jnp.ones((L,), jnp.float32)
    iota = lax.iota(jnp.int32, L)

    for d in range(D):
        idxm[d, pl.ds(0, L)] = (iota + d) & (D - 1)

    def zero_acc(k, carry):
        for j in range(D // L):
            acc[k, pl.ds(L * j, L)] = zeros
        cnt2[k, pl.ds(0, L)] = zeros
        return carry

    lax.fori_loop(0, C, zero_acc, 0)

    @plsc.parallel_loop(0, GROUPS, 1, unroll=2, carry=zeros)
    def sqfn(g, sqfn):
        col = pl.ds(g * L, L)
        lab = lvm[col]
        s0 = zeros
        s1 = zeros
        s2 = zeros
        s3 = zeros
        for d in range(0, D, 4):
            v0 = fvm[d, col]
            v1 = fvm[d + 1, col]
            v2 = fvm[d + 2, col]
            v3 = fvm[d + 3, col]
            s0 = s0 + v0 * v0
            s1 = s1 + v1 * v1
            s2 = s2 + v2 * v2
            s3 = s3 + v3 * v3
        sq = (s0 + s1) + (s2 + s3)
        inv = _rsqrt16(jnp.maximum(sq, 1e-24))
        sqfn = sqfn + sq * (inv * inv)
        for d in range(D):
            v = fvm[d, col] * inv
            plsc.addupdate_scatter(acc, [lab, jnp.full((L,), d, jnp.int32)], v)
        plsc.addupdate_scatter(cnt2, [lab, iota], ones)
        return sqfn

    sqv[pl.ds(0, L)] = sqfn

    pltpu.sync_copy(acc, sums_out.at[wid])
    pltpu.sync_copy(cnt2, cnt_out.at[wid])
    pltpu.sync_copy(sqv, sq_out.at[wid])


_sc_segment = pl.kernel(
    _sc_body,
    out_type=[
        jax.ShapeDtypeStruct((NW, C, D), jnp.float32),
        jax.ShapeDtypeStruct((NW, C, L), jnp.float32),
        jax.ShapeDtypeStruct((NW, L), jnp.float32),
    ],
    mesh=plsc.VectorSubcoreMesh(
        core_axis_name="c", subcore_axis_name="s",
        num_cores=NUM_CORES, num_subcores=NUM_SUBCORES,
    ),
    scratch_types=[
        pltpu.VMEM((D, RPW), jnp.float32),
        pltpu.VMEM((RPW,), jnp.int32),
        pltpu.VMEM((C, D), jnp.float32),
        pltpu.VMEM((C, L), jnp.float32),
        pltpu.VMEM((L,), jnp.float32),
        pltpu.VMEM((D, L), jnp.int32),
    ],
    compiler_params=pltpu.CompilerParams(needs_layout_passes=False),
)


def _epi_body(sums_ref, cnt_ref, sq_ref, out_ref):
    sums = jnp.sum(sums_ref[...], axis=0)
    counts = jnp.sum(jnp.sum(cnt_ref[...], axis=0), axis=1)
    sqtot = jnp.sum(sq_ref[...])
    csafe = jnp.maximum(counts, 1.0)
    centers = sums / csafe[:, None]
    cnorm2 = jnp.sum(centers * centers, axis=1)
    intra = (sqtot - jnp.sum(counts * cnorm2)) / jnp.float32(N_ROWS)
    gram = jnp.dot(centers, centers.T, preferred_element_type=jnp.float32)
    d2 = cnorm2[:, None] + cnorm2[None, :] - 2.0 * gram
    d2 = jnp.maximum(d2, 0.0)
    row = lax.broadcasted_iota(jnp.int32, (C, C), 0)
    col = lax.broadcasted_iota(jnp.int32, (C, C), 1)
    pres = counts > 0.5
    mask = (row != col) & pres[:, None] & pres[None, :]
    min_d2 = jnp.min(jnp.where(mask, d2, jnp.float32(1e30)))
    min_inter = jnp.sqrt(min_d2)
    inter = jnp.maximum(MARGIN_ - min_inter, 0.0)
    sr = jnp.clip(min_inter / MARGIN_, 0.0, 1.0)
    loss = (1.0 + 2.0 * (1.0 - sr)) * intra + (2.0 * sr) * inter
    npres = jnp.sum(pres.astype(jnp.float32))
    loss = jnp.where(npres < 1.5, jnp.float32(0.0), loss)
    out_ref[...] = jnp.broadcast_to(loss, (1, 1))


_epilogue = pl.pallas_call(
    _epi_body,
    out_shape=jax.ShapeDtypeStruct((1, 1), jnp.float32),
)


@jax.jit
def kernel(features, labels):
    fT = features.T
    sums, cnt, sq = _sc_segment(fT, labels)
    return _epilogue(sums, cnt, sq)[0, 0]


# speedup vs baseline: 1.2673x; 1.2673x over previous
import numpy as np

import jax
import jax.numpy as jnp
from jax import lax
from jax.experimental import pallas as pl
from jax.experimental.pallas import tpu as pltpu
from jax.experimental.pallas import tpu_sc as plsc

N_ROWS = 16384
D = 64
C = 64
MARGIN_ = 2.0

NUM_CORES = 2
NUM_SUBCORES = 16
NW = NUM_CORES * NUM_SUBCORES
RPW = N_ROWS // NW
L = 16
GROUPS = RPW // L

_RSQRT_MAGIC = np.int32(0x5F3759DF)


def _rsqrt16(s):
    bits = plsc.bitcast(s, jnp.int32)
    y = plsc.bitcast(_RSQRT_MAGIC - (bits >> 1), jnp.float32)
    for _ in range(3):
        y = y * (1.5 - 0.5 * s * y * y)
    return y


def _sc_body(fT, labels, sums_out, cnt_out, sq_out, fvm, lvm, acc, cnt2, sqv, invv):
    cid = lax.axis_index("c")
    sid = lax.axis_index("s")
    wid = sid * NUM_CORES + cid

    pltpu.sync_copy(fT.at[:, pl.ds(wid * RPW, RPW)], fvm)
    pltpu.sync_copy(labels.at[pl.ds(wid * RPW, RPW)], lvm)

    zeros = jnp.zeros((L,), jnp.float32)
    ones = jnp.ones((L,), jnp.float32)
    iota = lax.iota(jnp.int32, L)

    def zero_acc(k, carry):
        for j in range(D // L):
            acc[k, pl.ds(L * j, L)] = zeros
        cnt2[k, pl.ds(0, L)] = zeros
        return carry

    lax.fori_loop(0, C, zero_acc, 0)

    @plsc.parallel_loop(0, GROUPS, 1, unroll=2, carry=zeros)
    def sqfn(g, sqfn):
        col = pl.ds(g * L, L)
        s0 = zeros
        s1 = zeros
        s2 = zeros
        s3 = zeros
        for d in range(0, D, 4):
            v0 = fvm[d, col]
            v1 = fvm[d + 1, col]
            v2 = fvm[d + 2, col]
            v3 = fvm[d + 3, col]
            s0 = s0 + v0 * v0
            s1 = s1 + v1 * v1
            s2 = s2 + v2 * v2
            s3 = s3 + v3 * v3
        sq = (s0 + s1) + (s2 + s3)
        inv = _rsqrt16(jnp.maximum(sq, 1e-24))
        invv[col] = inv
        return sqfn + sq * (inv * inv)

    sqv[pl.ds(0, L)] = sqfn

    @plsc.parallel_loop(0, GROUPS, 1, unroll=2)
    def _(g):
        col = pl.ds(g * L, L)
        lab = lvm[col]
        inv = invv[col]
        rowv = iota + g * L
        for d in range(D):
            drot = (iota + d) & (D - 1)
            v = plsc.load_gather(fvm, [drot, rowv]) * inv
            plsc.addupdate_scatter(acc, [lab, drot], v)
        plsc.addupdate_scatter(cnt2, [lab, iota], ones)

    pltpu.sync_copy(acc, sums_out.at[wid])
    pltpu.sync_copy(cnt2, cnt_out.at[wid])
    pltpu.sync_copy(sqv, sq_out.at[wid])


_sc_segment = pl.kernel(
    _sc_body,
    out_type=[
        jax.ShapeDtypeStruct((NW, C, D), jnp.float32),
        jax.ShapeDtypeStruct((NW, C, L), jnp.float32),
        jax.ShapeDtypeStruct((NW, L), jnp.float32),
    ],
    mesh=plsc.VectorSubcoreMesh(
        core_axis_name="c", subcore_axis_name="s",
        num_cores=NUM_CORES, num_subcores=NUM_SUBCORES,
    ),
    scratch_types=[
        pltpu.VMEM((D, RPW), jnp.float32),
        pltpu.VMEM((RPW,), jnp.int32),
        pltpu.VMEM((C, D), jnp.float32),
        pltpu.VMEM((C, L), jnp.float32),
        pltpu.VMEM((L,), jnp.float32),
        pltpu.VMEM((RPW,), jnp.float32),
    ],
    compiler_params=pltpu.CompilerParams(needs_layout_passes=False),
)


def _epi_body(sums_ref, cnt_ref, sq_ref, out_ref):
    sums = jnp.sum(sums_ref[...], axis=0)
    counts = jnp.sum(jnp.sum(cnt_ref[...], axis=0), axis=1)
    sqtot = jnp.sum(sq_ref[...])
    csafe = jnp.maximum(counts, 1.0)
    centers = sums / csafe[:, None]
    cnorm2 = jnp.sum(centers * centers, axis=1)
    intra = (sqtot - jnp.sum(counts * cnorm2)) / jnp.float32(N_ROWS)
    gram = jnp.dot(centers, centers.T, preferred_element_type=jnp.float32)
    d2 = cnorm2[:, None] + cnorm2[None, :] - 2.0 * gram
    d2 = jnp.maximum(d2, 0.0)
    row = lax.broadcasted_iota(jnp.int32, (C, C), 0)
    col = lax.broadcasted_iota(jnp.int32, (C, C), 1)
    pres = counts > 0.5
    mask = (row != col) & pres[:, None] & pres[None, :]
    min_d2 = jnp.min(jnp.where(mask, d2, jnp.float32(1e30)))
    min_inter = jnp.sqrt(min_d2)
    inter = jnp.maximum(MARGIN_ - min_inter, 0.0)
    sr = jnp.clip(min_inter / MARGIN_, 0.0, 1.0)
    loss = (1.0 + 2.0 * (1.0 - sr)) * intra + (2.0 * sr) * inter
    npres = jnp.sum(pres.astype(jnp.float32))
    loss = jnp.where(npres < 1.5, jnp.float32(0.0), loss)
    out_ref[...] = jnp.broadcast_to(loss, (1, 1))


_epilogue = pl.pallas_call(
    _epi_body,
    out_shape=jax.ShapeDtypeStruct((1, 1), jnp.float32),
)


@jax.jit
def kernel(features, labels):
    fT = features.T
    sums, cnt, sq = _sc_segment(fT, labels)
    return _epilogue(sums, cnt, sq)[0, 0]


# speedup vs baseline: 1.4911x; 1.1766x over previous
import numpy as np

import jax
import jax.numpy as jnp
from jax import lax
from jax.experimental import pallas as pl
from jax.experimental.pallas import tpu as pltpu
from jax.experimental.pallas import tpu_sc as plsc

N_ROWS = 16384
D = 64
C = 64
MARGIN_ = 2.0

NUM_CORES = 2
NUM_SUBCORES = 16
NW = NUM_CORES * NUM_SUBCORES
RPW = N_ROWS // NW
L = 16
GROUPS = RPW // L

_RSQRT_MAGIC = np.int32(0x5F3759DF)


def _rsqrt16(s):
    bits = plsc.bitcast(s, jnp.int32)
    y = plsc.bitcast(_RSQRT_MAGIC - (bits >> 1), jnp.float32)
    for _ in range(3):
        y = y * (1.5 - 0.5 * s * y * y)
    return y


def _sc_body(fT, labels, sums_out, cnt_out, sq_out, fvm, lvm, acc, cnt2, sqv):
    cid = lax.axis_index("c")
    sid = lax.axis_index("s")
    wid = sid * NUM_CORES + cid

    pltpu.sync_copy(fT.at[:, pl.ds(wid * RPW, RPW)], fvm)
    pltpu.sync_copy(labels.at[pl.ds(wid * RPW, RPW)], lvm)

    zeros = jnp.zeros((L,), jnp.float32)
    ones = jnp.ones((L,), jnp.float32)
    iota = lax.iota(jnp.int32, L)

    def zero_acc(k, carry):
        for j in range(D // L):
            acc[k, pl.ds(L * j, L)] = zeros
        cnt2[k, pl.ds(0, L)] = zeros
        return carry

    lax.fori_loop(0, C, zero_acc, 0)

    @plsc.parallel_loop(0, GROUPS, 1, unroll=2, carry=zeros)
    def sqfn(g, sqfn):
        col = pl.ds(g * L, L)
        lab = lvm[col]
        s0 = zeros
        s1 = zeros
        s2 = zeros
        s3 = zeros
        for d in range(0, D, 4):
            v0 = fvm[d, col]
            v1 = fvm[d + 1, col]
            v2 = fvm[d + 2, col]
            v3 = fvm[d + 3, col]
            s0 = s0 + v0 * v0
            s1 = s1 + v1 * v1
            s2 = s2 + v2 * v2
            s3 = s3 + v3 * v3
        sq = (s0 + s1) + (s2 + s3)
        inv = _rsqrt16(jnp.maximum(sq, 1e-24))
        sqfn = sqfn + sq * (inv * inv)
        rowv = iota + g * L
        for d in range(D):
            drot = (iota + d) & (D - 1)
            v = plsc.load_gather(fvm, [drot, rowv]) * inv
            plsc.addupdate_scatter(acc, [lab, drot], v)
        plsc.addupdate_scatter(cnt2, [lab, iota], ones)
        return sqfn

    sqv[pl.ds(0, L)] = sqfn

    pltpu.sync_copy(acc, sums_out.at[wid])
    pltpu.sync_copy(cnt2, cnt_out.at[wid])
    pltpu.sync_copy(sqv, sq_out.at[wid])


_sc_segment = pl.kernel(
    _sc_body,
    out_type=[
        jax.ShapeDtypeStruct((NW, C, D), jnp.float32),
        jax.ShapeDtypeStruct((NW, C, L), jnp.float32),
        jax.ShapeDtypeStruct((NW, L), jnp.float32),
    ],
    mesh=plsc.VectorSubcoreMesh(
        core_axis_name="c", subcore_axis_name="s",
        num_cores=NUM_CORES, num_subcores=NUM_SUBCORES,
    ),
    scratch_types=[
        pltpu.VMEM((D, RPW), jnp.float32),
        pltpu.VMEM((RPW,), jnp.int32),
        pltpu.VMEM((C, D), jnp.float32),
        pltpu.VMEM((C, L), jnp.float32),
        pltpu.VMEM((L,), jnp.float32),
    ],
    compiler_params=pltpu.CompilerParams(needs_layout_passes=False),
)


def _epi_body(sums_ref, cnt_ref, sq_ref, out_ref):
    sums = jnp.sum(sums_ref[...], axis=0)
    counts = jnp.sum(jnp.sum(cnt_ref[...], axis=0), axis=1)
    sqtot = jnp.sum(sq_ref[...])
    csafe = jnp.maximum(counts, 1.0)
    centers = sums / csafe[:, None]
    cnorm2 = jnp.sum(centers * centers, axis=1)
    intra = (sqtot - jnp.sum(counts * cnorm2)) / jnp.float32(N_ROWS)
    gram = jnp.dot(centers, centers.T, preferred_element_type=jnp.float32)
    d2 = cnorm2[:, None] + cnorm2[None, :] - 2.0 * gram
    d2 = jnp.maximum(d2, 0.0)
    row = lax.broadcasted_iota(jnp.int32, (C, C), 0)
    col = lax.broadcasted_iota(jnp.int32, (C, C), 1)
    pres = counts > 0.5
    mask = (row != col) & pres[:, None] & pres[None, :]
    min_d2 = jnp.min(jnp.where(mask, d2, jnp.float32(1e30)))
    min_inter = jnp.sqrt(min_d2)
    inter = jnp.maximum(MARGIN_ - min_inter, 0.0)
    sr = jnp.clip(min_inter / MARGIN_, 0.0, 1.0)
    loss = (1.0 + 2.0 * (1.0 - sr)) * intra + (2.0 * sr) * inter
    npres = jnp.sum(pres.astype(jnp.float32))
    loss = jnp.where(npres < 1.5, jnp.float32(0.0), loss)
    out_ref[...] = jnp.broadcast_to(loss, (1, 1))


_epilogue = pl.pallas_call(
    _epi_body,
    out_shape=jax.ShapeDtypeStruct((1, 1), jnp.float32),
)


@jax.jit
def kernel(features, labels):
    fT = features.T
    sums, cnt, sq = _sc_segment(fT, labels)
    return _epilogue(sums, cnt, sq)[0, 0]
